# baseline (speedup 1.0000x reference)
import functools

import jax
import jax.numpy as jnp
from jax import lax
from jax.experimental import pallas as pl
from jax.experimental.pallas import tpu as pltpu
from jax.experimental.pallas import tpu_sc as plsc

_B = 32
_N = 100000
_L = 16
_CHUNK = 10000
_NCHUNK = _N // _CHUNK
_BIG = 1e10


def _argmax_update(vals, idx, best_v, best_i):
    upd = vals > best_v
    return jnp.where(upd, vals, best_v), jnp.where(upd, idx, best_i)


def _finalize_argmax(best_v, best_i):
    m = jnp.float32(-_BIG)
    im = jnp.int32(_N)
    for l in range(_L):
        v = best_v[l]
        ii = best_i[l]
        take = (v > m) | ((v == m) & (ii < im))
        m = jnp.where(take, v, m)
        im = jnp.where(take, ii, im)
    return im


def _lane(v, k):
    r = v[0]
    for l in range(1, _L):
        r = jnp.where(k == l, v[l], r)
    return r


def _fps_body(x_hbm, out_hbm, yv, xc, zc, win1, win2, st):
    nc = 2
    b = lax.axis_index("s") * nc + lax.axis_index("c")
    lane = lax.iota(jnp.int32, _L)

    pltpu.sync_copy(x_hbm.at[b, 1, :], yv)

    def body_a(i, carry):
        bv, bi = carry
        vals = yv[pl.ds(i * _L, _L)]
        return _argmax_update(vals, lane + i * _L, bv, bi)

    bv0 = jnp.full((_L,), -_BIG, jnp.float32)
    bi0 = jnp.zeros((_L,), jnp.int32)
    bv, bi = lax.fori_loop(0, _N // _L, body_a, (bv0, bi0))
    i0 = _finalize_argmax(bv, bi)

    base = pl.multiple_of(jnp.minimum((i0 // 8) * 8, _N - _L), 8)
    off = i0 - base
    pltpu.sync_copy(x_hbm.at[b, 0, pl.ds(base, _L)], win1)
    pltpu.sync_copy(x_hbm.at[b, 2, pl.ds(base, _L)], win2)
    cx = jnp.full((_L,), _lane(win1[...], off), jnp.float32)
    cz = jnp.full((_L,), _lane(win2[...], off), jnp.float32)
    cy = jnp.full((_L,), _lane(yv[pl.ds(base, _L)], off), jnp.float32)

    bv = jnp.full((_L,), -_BIG, jnp.float32)
    bi = jnp.zeros((_L,), jnp.int32)
    for j in range(_NCHUNK):
        pltpu.sync_copy(x_hbm.at[b, 0, pl.ds(j * _CHUNK, _CHUNK)], xc)
        pltpu.sync_copy(x_hbm.at[b, 2, pl.ds(j * _CHUNK, _CHUNK)], zc)

        def body_b(i, carry, j=j):
            cbv, cbi = carry
            vx = xc[pl.ds(i * _L, _L)]
            vz = zc[pl.ds(i * _L, _L)]
            vy = yv[pl.ds(j * _CHUNK + i * _L, _L)]
            dx = vx - cx
            dy = vy - cy
            dz = vz - cz
            d = dx * dx + dy * dy + dz * dz
            return _argmax_update(d, lane + (j * _CHUNK + i * _L), cbv, cbi)

        bv, bi = lax.fori_loop(0, _CHUNK // _L, body_b, (bv, bi))
    i1 = _finalize_argmax(bv, bi)

    res = jnp.where(lane == 0, i0, jnp.where(lane == 1, i1, 0))
    st[...] = res.reshape(1, _L)
    pltpu.sync_copy(st, out_hbm.at[b])


def kernel(xyz):
    x = xyz.reshape(_B, 3, _N)
    mesh = plsc.VectorSubcoreMesh(core_axis_name="c", subcore_axis_name="s")
    fps = functools.partial(
        pl.kernel,
        mesh=mesh,
        out_type=jax.ShapeDtypeStruct((_B, 1, _L), jnp.int32),
        scratch_types=[
            pltpu.VMEM((_N,), jnp.float32),
            pltpu.VMEM((_CHUNK,), jnp.float32),
            pltpu.VMEM((_CHUNK,), jnp.float32),
            pltpu.VMEM((_L,), jnp.float32),
            pltpu.VMEM((_L,), jnp.float32),
            pltpu.VMEM((1, _L), jnp.int32),
        ],
        compiler_params=pltpu.CompilerParams(use_tc_tiling_on_sc=False),
    )(_fps_body)
    out = fps(x)
    return out[:, 0, :2]


# speedup vs baseline: 4.6995x; 4.6995x over previous
import functools

import jax
import jax.numpy as jnp
from jax import lax
from jax.experimental import pallas as pl
from jax.experimental.pallas import tpu as pltpu
from jax.experimental.pallas import tpu_sc as plsc

_B = 32
_N = 100000
_L = 16
_CHUNK = 10240
_NMAIN = 9
_TAIL = _N - _NMAIN * _CHUNK
_UNROLL = 8
_BIG = 1e10


def _argmax_update(vals, idx, best_v, best_i):
    upd = vals > best_v
    return jnp.where(upd, vals, best_v), jnp.where(upd, idx, best_i)


def _finalize_argmax(best_v, best_i):
    m = jnp.float32(-_BIG)
    im = jnp.int32(_N)
    for l in range(_L):
        v = best_v[l]
        ii = best_i[l]
        take = (v > m) | ((v == m) & (ii < im))
        m = jnp.where(take, v, m)
        im = jnp.where(take, ii, im)
    return im


def _lane(v, k):
    r = v[0]
    for l in range(1, _L):
        r = jnp.where(k == l, v[l], r)
    return r


def _chunk_plan(bufs, buft):
    plan = []
    for j in range(_NMAIN):
        plan.append((j * _CHUNK, _CHUNK, bufs[j % 2]))
    plan.append((_NMAIN * _CHUNK, _TAIL, buft))
    return plan


def _streamed_pass(x_hbm, b, plan, sems, compute_chunk, carry):
    n = len(plan)

    def start(j):
        off, ln, dst = plan[j]
        pltpu.async_copy(x_hbm.at[b, :, pl.ds(off, ln)], dst, sems[j % 3])

    start(0)
    if n > 1:
        start(1)
    for j in range(n):
        off, ln, dst = plan[j]
        pltpu.make_async_copy(x_hbm.at[b, :, pl.ds(off, ln)], dst, sems[j % 3]).wait()
        carry = compute_chunk(off, ln, dst, carry)
        if j + 2 < n:
            start(j + 2)
    return carry


def _fps_body(x_hbm, out_hbm, buf0, buf1, buft, wv, st, sem0, sem1, sem2):
    nc = 2
    b = lax.axis_index("s") * nc + lax.axis_index("c")
    lane = lax.iota(jnp.int32, _L)
    plan = _chunk_plan((buf0, buf1), buft)
    sems = (sem0, sem1, sem2)

    def compute_a(off, ln, dst, carry):
        def body(i, c):
            vals = dst[1, pl.ds(i * _L, _L)]
            return _argmax_update(vals, lane + (off + i * _L), *c)

        return lax.fori_loop(0, ln // _L, body, carry, unroll=_UNROLL)

    bv0 = jnp.full((_L,), -_BIG, jnp.float32)
    bi0 = jnp.zeros((_L,), jnp.int32)
    bv, bi = _streamed_pass(x_hbm, b, plan, sems, compute_a, (bv0, bi0))
    i0 = _finalize_argmax(bv, bi)

    wbase = pl.multiple_of((i0 // 128) * 128, 128)
    pltpu.sync_copy(x_hbm.at[b, :, pl.ds(wbase, 128)], wv)
    woff = i0 - wbase
    w8 = pl.multiple_of(jnp.minimum((woff // 8) * 8, 128 - _L), 8)
    wk = woff - w8
    cx = jnp.full((_L,), _lane(wv[0, pl.ds(w8, _L)], wk), jnp.float32)
    cy = jnp.full((_L,), _lane(wv[1, pl.ds(w8, _L)], wk), jnp.float32)
    cz = jnp.full((_L,), _lane(wv[2, pl.ds(w8, _L)], wk), jnp.float32)

    def compute_b(off, ln, dst, carry):
        def body(i, c):
            vx = dst[0, pl.ds(i * _L, _L)]
            vy = dst[1, pl.ds(i * _L, _L)]
            vz = dst[2, pl.ds(i * _L, _L)]
            dx = vx - cx
            dy = vy - cy
            dz = vz - cz
            d = dx * dx + dy * dy + dz * dz
            return _argmax_update(d, lane + (off + i * _L), *c)

        return lax.fori_loop(0, ln // _L, body, carry, unroll=_UNROLL)

    bv, bi = _streamed_pass(x_hbm, b, plan, sems, compute_b, (bv0, bi0))
    i1 = _finalize_argmax(bv, bi)

    res = jnp.where(lane == 0, i0, jnp.where(lane == 1, i1, 0))
    st[...] = res.reshape(1, _L)
    pltpu.sync_copy(st, out_hbm.at[b])


def kernel(xyz):
    x = xyz.reshape(_B, 3, _N)
    mesh = plsc.VectorSubcoreMesh(core_axis_name="c", subcore_axis_name="s")
    fps = functools.partial(
        pl.kernel,
        mesh=mesh,
        out_type=jax.ShapeDtypeStruct((_B, 1, _L), jnp.int32),
        scratch_types=[
            pltpu.VMEM((3, _CHUNK), jnp.float32),
            pltpu.VMEM((3, _CHUNK), jnp.float32),
            pltpu.VMEM((3, _TAIL), jnp.float32),
            pltpu.VMEM((3, 128), jnp.float32),
            pltpu.VMEM((1, _L), jnp.int32),
            pltpu.SemaphoreType.DMA,
            pltpu.SemaphoreType.DMA,
            pltpu.SemaphoreType.DMA,
        ],
    )(_fps_body)
    out = fps(x)
    return out[:, 0, :2]


# speedup vs baseline: 4.7333x; 1.0072x over previous
import functools

import jax
import jax.numpy as jnp
from jax import lax
from jax.experimental import pallas as pl
from jax.experimental.pallas import tpu as pltpu
from jax.experimental.pallas import tpu_sc as plsc

_B = 32
_N = 100000
_L = 16
_CHUNK = 8192
_NMAIN = 12
_TAIL = _N - _NMAIN * _CHUNK
_UNROLL = 8
_BIG = 1e10


def _argmax_update(vals, idx, best_v, best_i):
    upd = vals > best_v
    return jnp.where(upd, vals, best_v), jnp.where(upd, idx, best_i)


def _finalize_argmax(best_v, best_i):
    m = jnp.float32(-_BIG)
    im = jnp.int32(_N)
    for l in range(_L):
        v = best_v[l]
        ii = best_i[l]
        take = (v > m) | ((v == m) & (ii < im))
        m = jnp.where(take, v, m)
        im = jnp.where(take, ii, im)
    return im


def _lane(v, k):
    r = v[0]
    for l in range(1, _L):
        r = jnp.where(k == l, v[l], r)
    return r


def _fps_body(x_hbm, out_hbm, buf0, buf1, buf2, buft, wv, st, s0, s1, s2, st_sem):
    nc = 2
    b = lax.axis_index("s") * nc + lax.axis_index("c")
    lane = lax.iota(jnp.int32, _L)

    rbufs = (buf0, buf1, buf2)
    rsems = (s0, s1, s2)
    pass_slots = [
        (j * _CHUNK, _CHUNK, rbufs[j % 3], rsems[j % 3]) for j in range(_NMAIN)
    ] + [(_NMAIN * _CHUNK, _TAIL, buft, st_sem)]
    slots = pass_slots + pass_slots
    nslots = len(slots)
    boundary = len(pass_slots)

    def start(j):
        off, ln, dst, sem = slots[j]
        pltpu.async_copy(x_hbm.at[b, :, pl.ds(off, ln)], dst, sem)

    def wait(j):
        off, ln, dst, sem = slots[j]
        pltpu.make_async_copy(x_hbm.at[b, :, pl.ds(off, ln)], dst, sem).wait()

    def compute_a(off, ln, dst, carry):
        def body(i, c):
            vals = dst[1, pl.ds(i * _L, _L)]
            return _argmax_update(vals, lane + (off + i * _L), *c)

        return lax.fori_loop(0, ln // _L, body, carry, unroll=_UNROLL)

    def make_compute_b(c3):
        cx, cy, cz = c3

        def compute_b(off, ln, dst, carry):
            def body(i, c):
                vx = dst[0, pl.ds(i * _L, _L)]
                vy = dst[1, pl.ds(i * _L, _L)]
                vz = dst[2, pl.ds(i * _L, _L)]
                dx = vx - cx
                dy = vy - cy
                dz = vz - cz
                d = dx * dx + dy * dy + dz * dz
                return _argmax_update(d, lane + (off + i * _L), *c)

            return lax.fori_loop(0, ln // _L, body, carry, unroll=_UNROLL)

        return compute_b

    zero_carry = (
        jnp.full((_L,), -_BIG, jnp.float32),
        jnp.zeros((_L,), jnp.int32),
    )

    start(0)
    start(1)
    carry = zero_carry
    i0 = None
    compute = compute_a
    for j in range(nslots):
        if j == boundary:
            i0 = _finalize_argmax(*carry)
            wbase = pl.multiple_of((i0 // 128) * 128, 128)
            pltpu.sync_copy(x_hbm.at[b, :, pl.ds(wbase, 128)], wv)
            woff = i0 - wbase
            w8 = pl.multiple_of(jnp.minimum((woff // 8) * 8, 128 - _L), 8)
            wk = woff - w8
            c3 = tuple(
                jnp.full((_L,), _lane(wv[r, pl.ds(w8, _L)], wk), jnp.float32)
                for r in range(3)
            )
            compute = make_compute_b(c3)
            carry = zero_carry
        off, ln, dst, _ = slots[j]
        wait(j)
        carry = compute(off, ln, dst, carry)
        if j + 2 < nslots:
            start(j + 2)
    i1 = _finalize_argmax(*carry)

    res = jnp.where(lane == 0, i0, jnp.where(lane == 1, i1, 0))
    st[...] = res.reshape(1, _L)
    pltpu.sync_copy(st, out_hbm.at[b])


def kernel(xyz):
    x = xyz.reshape(_B, 3, _N)
    mesh = plsc.VectorSubcoreMesh(core_axis_name="c", subcore_axis_name="s")
    fps = functools.partial(
        pl.kernel,
        mesh=mesh,
        out_type=jax.ShapeDtypeStruct((_B, 1, _L), jnp.int32),
        scratch_types=[
            pltpu.VMEM((3, _CHUNK), jnp.float32),
            pltpu.VMEM((3, _CHUNK), jnp.float32),
            pltpu.VMEM((3, _CHUNK), jnp.float32),
            pltpu.VMEM((3, _TAIL), jnp.float32),
            pltpu.VMEM((3, 128), jnp.float32),
            pltpu.VMEM((1, _L), jnp.int32),
            pltpu.SemaphoreType.DMA,
            pltpu.SemaphoreType.DMA,
            pltpu.SemaphoreType.DMA,
            pltpu.SemaphoreType.DMA,
        ],
    )(_fps_body)
    out = fps(x)
    return out[:, 0, :2]
